# baseline (speedup 1.0000x reference)
import functools

import jax
import jax.numpy as jnp
from jax import lax
from jax.experimental import pallas as pl
from jax.experimental.pallas import tpu as pltpu
from jax.experimental.pallas import tpu_sc as plsc

N = 10000
NP = 10240
E = 320000
D = 128
NC = 2
NS = 16
NW = NC * NS
CH = 128

NCH0 = 80
EP0 = NW * NCH0 * CH
TRASH0 = N + 16

NCH1 = 160
EP1 = NS * NCH1 * CH
QB = NP // 4
ACC_R = QB + 128
RPT0 = NP // NS
RPT1 = QB // NS

_mesh = plsc.VectorSubcoreMesh(core_axis_name="c", subcore_axis_name="s")


@functools.partial(
    pl.kernel,
    out_type=jax.ShapeDtypeStruct((NC, NP), jnp.float32),
    mesh=_mesh,
    scratch_types=[
        pltpu.VMEM((NCH0, CH), jnp.int32),
        pltpu.VMEM((CH,), jnp.float32),
        pltpu.VMEM((RPT0,), jnp.float32),
        pltpu.VMEM_SHARED((NP,), jnp.float32),
    ],
)
def _deg_kernel(dst_hbm, out_hbm, idx_v, ones_v, stage_v, deg_sp):
    c = lax.axis_index("c")
    s = lax.axis_index("s")
    w = c * NS + s

    def fill_ones(i, carry):
        ones_v[pl.ds(i * 16, 16)] = jnp.ones((16,), jnp.float32)
        return carry

    lax.fori_loop(0, CH // 16, fill_ones, 0)

    def fill_zero(i, carry):
        stage_v[pl.ds(i * 16, 16)] = jnp.zeros((16,), jnp.float32)
        return carry

    lax.fori_loop(0, RPT0 // 16, fill_zero, 0)

    pltpu.sync_copy(stage_v, deg_sp.at[pl.ds(s * RPT0, RPT0)])
    pltpu.sync_copy(dst_hbm.at[w], idx_v)
    plsc.subcore_barrier()

    def chunk(j, carry):
        pltpu.sync_copy(ones_v, deg_sp.at[idx_v.at[j]], add=True)
        return carry

    lax.fori_loop(0, NCH0, chunk, 0)

    plsc.subcore_barrier()
    pltpu.sync_copy(deg_sp.at[pl.ds(s * RPT0, RPT0)], stage_v)
    pltpu.sync_copy(stage_v, out_hbm.at[c, pl.ds(s * RPT0, RPT0)])


NQ = NCH1 // 4


@functools.partial(
    pl.kernel,
    out_type=jax.ShapeDtypeStruct((NC, QB, D), jnp.float32),
    mesh=_mesh,
    scratch_types=[
        pltpu.VMEM((NCH1, CH), jnp.int32),
        pltpu.VMEM((NCH1, CH), jnp.int32),
        pltpu.VMEM((CH, D), jnp.float32),
        pltpu.VMEM((CH, D), jnp.float32),
        pltpu.VMEM((CH, D), jnp.float32),
        pltpu.VMEM((CH, D), jnp.float32),
        pltpu.SemaphoreType.DMA,
        pltpu.SemaphoreType.DMA,
        pltpu.SemaphoreType.DMA,
        pltpu.SemaphoreType.DMA,
        pltpu.SemaphoreType.DMA,
        pltpu.SemaphoreType.DMA,
        pltpu.SemaphoreType.DMA,
        pltpu.SemaphoreType.DMA,
        pltpu.VMEM_SHARED((ACC_R, D), jnp.float32),
    ],
)
def _agg_kernel(src_hbm, dstl_hbm, hn_hbm, out_hbm,
                srcv, dstv, buf0, buf1, buf2, buf3,
                gs0, gs1, gs2, gs3, ss0, ss1, ss2, ss3, agg_sp):
    c = lax.axis_index("c")
    s = lax.axis_index("s")
    bufs = (buf0, buf1, buf2, buf3)
    gsems = (gs0, gs1, gs2, gs3)
    ssems = (ss0, ss1, ss2, ss3)

    pltpu.sync_copy(src_hbm.at[s], srcv)
    pltpu.sync_copy(dstl_hbm.at[c, s], dstv)

    def zrow(r, carry):
        def zcol(k, inner):
            buf0[r, pl.ds(k * 16, 16)] = jnp.zeros((16,), jnp.float32)
            return inner

        lax.fori_loop(0, D // 16, zcol, 0)
        return carry

    lax.fori_loop(0, CH, zrow, 0)
    pltpu.sync_copy(buf0, agg_sp.at[pl.ds(s * RPT1, CH)])
    pltpu.sync_copy(buf0.at[pl.ds(0, RPT1 - CH)],
                    agg_sp.at[pl.ds(s * RPT1 + CH, RPT1 - CH)])
    plsc.subcore_barrier()

    for i in range(4):
        pltpu.async_copy(hn_hbm.at[srcv.at[i]], bufs[i], gsems[i])

    def quad(q, carry):
        j = q * 4
        for i in range(4):
            pltpu.make_async_copy(
                hn_hbm.at[srcv.at[j + i]], bufs[i], gsems[i]).wait()
            pltpu.async_copy(
                bufs[i], agg_sp.at[dstv.at[j + i]], ssems[i], add=True)

        @pl.when(q + 1 < NQ)
        def _():
            for i in range(4):
                pltpu.make_async_copy(
                    bufs[i], agg_sp.at[dstv.at[j + i]], ssems[i]).wait()
                pltpu.async_copy(
                    hn_hbm.at[srcv.at[j + 4 + i]], bufs[i], gsems[i])

        return carry

    lax.fori_loop(0, NQ, quad, 0)

    for i in range(4):
        pltpu.make_async_copy(
            bufs[i], agg_sp.at[dstv.at[(NQ - 1) * 4 + i]], ssems[i]).wait()

    plsc.subcore_barrier()

    pltpu.sync_copy(agg_sp.at[pl.ds(s * RPT1, CH)], buf0)
    pltpu.sync_copy(buf0, out_hbm.at[c, pl.ds(s * RPT1, CH)])
    pltpu.sync_copy(agg_sp.at[pl.ds(s * RPT1 + CH, RPT1 - CH)],
                    buf1.at[pl.ds(0, RPT1 - CH)])
    pltpu.sync_copy(buf1.at[pl.ds(0, RPT1 - CH)],
                    out_hbm.at[c, pl.ds(s * RPT1 + CH, RPT1 - CH)])


BLK = 1024


def _dense_body(x_ref, w_ref, dp_ref, hn_ref):
    deg = dp_ref[0, :] + dp_ref[1, :] + 1.0
    dinv = lax.rsqrt(deg)
    h = jnp.dot(x_ref[...], w_ref[...], preferred_element_type=jnp.float32)
    hn_ref[...] = h * dinv[:, None]


_dense = pl.pallas_call(
    _dense_body,
    grid=(NP // BLK,),
    in_specs=[
        pl.BlockSpec((BLK, D), lambda i: (i, 0)),
        pl.BlockSpec((D, D), lambda i: (0, 0)),
        pl.BlockSpec((NC, BLK), lambda i: (0, i)),
    ],
    out_specs=pl.BlockSpec((BLK, D), lambda i: (i, 0)),
    out_shape=jax.ShapeDtypeStruct((NP, D), jnp.float32),
)


def _final_body(a_ref, hn_ref, dp_ref, b_ref, o_ref):
    deg = dp_ref[0, :] + dp_ref[1, :] + 1.0
    dinv = lax.rsqrt(deg)
    ssum = a_ref[...] + hn_ref[...]
    o_ref[...] = jnp.maximum(ssum * dinv[:, None] + b_ref[...][None, :], 0.0)


_final = pl.pallas_call(
    _final_body,
    grid=(NP // BLK,),
    in_specs=[
        pl.BlockSpec((BLK, D), lambda i: (i, 0)),
        pl.BlockSpec((BLK, D), lambda i: (i, 0)),
        pl.BlockSpec((NC, BLK), lambda i: (0, i)),
        pl.BlockSpec((D,), lambda i: (0,)),
    ],
    out_specs=pl.BlockSpec((BLK, D), lambda i: (i, 0)),
    out_shape=jax.ShapeDtypeStruct((NP, D), jnp.float32),
)


def kernel(x, index, W, b):
    idx32 = index.astype(jnp.int32)
    src = idx32[0]
    dst = idx32[1]

    dst0 = jnp.concatenate(
        [dst, jnp.full((EP0 - E,), TRASH0, jnp.int32)]).reshape(NW, NCH0, CH)

    src1 = jnp.concatenate(
        [src, jnp.zeros((EP1 - E,), jnp.int32)]).reshape(NS, NCH1, CH)
    dstp = jnp.concatenate([dst, jnp.full((EP1 - E,), -1, jnp.int32)])
    los = jnp.arange(NC, dtype=jnp.int32) * (NP // NC)
    trash = QB + (jnp.arange(EP1, dtype=jnp.int32) % 128)
    dstls = []
    for k in range(2):
        local = dstp[None, :] - (los[:, None] + k * QB)
        dstls.append(jnp.where(
            (local >= 0) & (local < QB), local,
            trash[None, :]).reshape(NC, NS, NCH1, CH))

    xp = jnp.pad(x, ((0, NP - N), (0, 0)))

    degp = _deg_kernel(dst0)
    hn = _dense(xp, W, degp)
    agg0 = _agg_kernel(src1, dstls[0], hn)
    agg1 = _agg_kernel(src1, dstls[1], hn)
    aggf = jnp.concatenate(
        [agg0[:, None], agg1[:, None]], axis=1).reshape(NP, D)
    out = _final(aggf, hn, degp, b)
    return (out[:N], index)


# speedup vs baseline: 2.7979x; 2.7979x over previous
import functools

import jax
import jax.numpy as jnp
import numpy as np
from jax import lax
from jax.experimental import pallas as pl
from jax.experimental.pallas import tpu as pltpu
from jax.experimental.pallas import tpu_sc as plsc

N = 10000
NP = 10240
E = 320000
D = 128
DH = D // 2
NC = 2
NS = 16
NW = NC * NS
CH = 128

NCH0 = 80
EP0 = NW * NCH0 * CH
TRASH0 = N + 16
RPT0 = NP // NS

NCH1 = 158
EP1 = NS * NCH1 * CH
HB = NP // NC
ACC_R = HB + 128
RPT1 = HB // NS

_mesh = plsc.VectorSubcoreMesh(core_axis_name="c", subcore_axis_name="s")


@functools.partial(
    pl.kernel,
    out_type=jax.ShapeDtypeStruct((NC, NP), jnp.float32),
    mesh=_mesh,
    scratch_types=[
        pltpu.VMEM((NCH0, CH), jnp.int32),
        pltpu.VMEM((CH,), jnp.float32),
        pltpu.VMEM((RPT0,), jnp.float32),
        pltpu.VMEM_SHARED((NP,), jnp.float32),
    ],
)
def _deg_kernel(dst_hbm, out_hbm, idx_v, ones_v, stage_v, deg_sp):
    c = lax.axis_index("c")
    s = lax.axis_index("s")
    w = c * NS + s

    def fill_ones(i, carry):
        ones_v[pl.ds(i * 16, 16)] = jnp.ones((16,), jnp.float32)
        return carry

    lax.fori_loop(0, CH // 16, fill_ones, 0)

    def fill_zero(i, carry):
        stage_v[pl.ds(i * 16, 16)] = jnp.zeros((16,), jnp.float32)
        return carry

    lax.fori_loop(0, RPT0 // 16, fill_zero, 0)

    pltpu.sync_copy(stage_v, deg_sp.at[pl.ds(s * RPT0, RPT0)])
    pltpu.sync_copy(dst_hbm.at[w], idx_v)
    plsc.subcore_barrier()

    def chunk(j, carry):
        pltpu.sync_copy(ones_v, deg_sp.at[idx_v.at[j]], add=True)
        return carry

    lax.fori_loop(0, NCH0, chunk, 0)

    plsc.subcore_barrier()
    pltpu.sync_copy(deg_sp.at[pl.ds(s * RPT0, RPT0)], stage_v)
    pltpu.sync_copy(stage_v, out_hbm.at[c, pl.ds(s * RPT0, RPT0)])


@functools.partial(
    pl.kernel,
    out_type=jax.ShapeDtypeStruct((NC, HB // 2, D), jnp.int32),
    mesh=_mesh,
    scratch_types=[
        pltpu.VMEM((NCH1, CH), jnp.int32),
        pltpu.VMEM((NCH1, CH), jnp.int32),
        pltpu.VMEM((CH, D), jnp.float32),
        pltpu.VMEM((CH, D), jnp.float32),
        pltpu.VMEM((CH // 2, D), jnp.int32),
        pltpu.VMEM_SHARED((ACC_R, D), jnp.float32),
        pltpu.SemaphoreType.DMA,
        pltpu.SemaphoreType.DMA,
    ],
)
def _agg_kernel(src_hbm, dstl_hbm, hnp_hbm, out_hbm,
                srcv, dstv, buf0, buf1, pbuf, agg_sp, sem0, sem1):
    c = lax.axis_index("c")
    s = lax.axis_index("s")

    pltpu.sync_copy(src_hbm.at[s], srcv)
    pltpu.sync_copy(dstl_hbm.at[c, s], dstv)

    def zrow(r, carry):
        def zcol(k, inner):
            buf0[r, pl.ds(k * 16, 16)] = jnp.zeros((16,), jnp.float32)
            return inner

        lax.fori_loop(0, D // 16, zcol, 0)
        return carry

    lax.fori_loop(0, CH, zrow, 0)
    pltpu.sync_copy(buf0, agg_sp.at[pl.ds(s * RPT1, CH)])
    pltpu.sync_copy(buf0, agg_sp.at[pl.ds(s * RPT1 + CH, CH)])
    pltpu.sync_copy(buf0.at[pl.ds(0, RPT1 - 2 * CH)],
                    agg_sp.at[pl.ds(s * RPT1 + 2 * CH, RPT1 - 2 * CH)])
    plsc.subcore_barrier()

    pltpu.async_copy(hnp_hbm.at[srcv.at[0]], buf0, sem0)

    def pair(p, carry):
        j0 = p * 2
        pltpu.make_async_copy(hnp_hbm.at[srcv.at[j0]], buf0, sem0).wait()
        pltpu.async_copy(hnp_hbm.at[srcv.at[j0 + 1]], buf1, sem1)
        pltpu.sync_copy(buf0, agg_sp.at[dstv.at[j0]], add=True)
        pltpu.make_async_copy(
            hnp_hbm.at[srcv.at[j0 + 1]], buf1, sem1).wait()

        @pl.when(p + 1 < NCH1 // 2)
        def _():
            pltpu.async_copy(hnp_hbm.at[srcv.at[j0 + 2]], buf0, sem0)

        pltpu.sync_copy(buf1, agg_sp.at[dstv.at[j0 + 1]], add=True)
        return carry

    lax.fori_loop(0, NCH1 // 2, pair, 0)

    plsc.subcore_barrier()

    buf0i = buf0.bitcast(jnp.int32)

    for off, nrows in ((0, CH), (CH, CH), (2 * CH, RPT1 - 2 * CH)):
        pltpu.sync_copy(
            agg_sp.at[pl.ds(s * RPT1 + off, nrows)],
            buf0.at[pl.ds(0, nrows)])

        def prow(r, carry):
            for half in range(2):
                for g in range(4):
                    ai = buf0i[2 * r + half, pl.ds(32 * g, 16)]
                    bi = buf0i[2 * r + half, pl.ds(32 * g + 16, 16)]
                    ar = ai + 0x7FFF + ((ai >> 16) & 1)
                    br = bi + 0x7FFF + ((bi >> 16) & 1)
                    w = ((ar >> 16) & 0xFFFF) | (br & jnp.int32(-65536))
                    pbuf[r, pl.ds(64 * half + 16 * g, 16)] = w
            return carry

        lax.fori_loop(0, nrows // 2, prow, 0)
        pltpu.sync_copy(
            pbuf.at[pl.ds(0, nrows // 2)],
            out_hbm.at[c, pl.ds(s * (RPT1 // 2) + off // 2, nrows // 2)])


BLK = 1024


def _dense_body(x_ref, w_ref, wp_ref, dp_ref, hn_ref, hnp_ref):
    deg = dp_ref[0, :] + dp_ref[1, :] + 1.0
    dinv = lax.rsqrt(deg)
    h = jnp.dot(x_ref[...], w_ref[...], preferred_element_type=jnp.float32)
    hn_ref[...] = h * dinv[:, None]
    hp = jnp.dot(x_ref[...], wp_ref[...], preferred_element_type=jnp.float32)
    hnp_ref[...] = hp * dinv[:, None]


_dense = pl.pallas_call(
    _dense_body,
    grid=(NP // BLK,),
    in_specs=[
        pl.BlockSpec((BLK, D), lambda i: (i, 0)),
        pl.BlockSpec((D, D), lambda i: (0, 0)),
        pl.BlockSpec((D, D), lambda i: (0, 0)),
        pl.BlockSpec((NC, BLK), lambda i: (0, i)),
    ],
    out_specs=[
        pl.BlockSpec((BLK, D), lambda i: (i, 0)),
        pl.BlockSpec((BLK, D), lambda i: (i, 0)),
    ],
    out_shape=[
        jax.ShapeDtypeStruct((NP, D), jnp.float32),
        jax.ShapeDtypeStruct((NP, D), jnp.float32),
    ],
)


def _final_body(a_ref, hn_ref, dp_ref, b_ref, o_ref):
    deg = dp_ref[0, :] + dp_ref[1, :] + 1.0
    dinv = lax.rsqrt(deg)
    ai = a_ref[0]
    lo = lax.bitcast_convert_type(ai << 16, jnp.float32)
    hi = lax.bitcast_convert_type(ai & jnp.int32(-65536), jnp.float32)
    even = jnp.concatenate([lo[:, :DH], hi[:, :DH]], axis=-1)
    odd = jnp.concatenate([lo[:, DH:], hi[:, DH:]], axis=-1)
    agg = jnp.stack([even, odd], axis=1).reshape(BLK, D)
    ssum = agg + hn_ref[...]
    o_ref[...] = jnp.maximum(ssum * dinv[:, None] + b_ref[...][None, :], 0.0)


_final = pl.pallas_call(
    _final_body,
    grid=(NP // BLK,),
    in_specs=[
        pl.BlockSpec((1, BLK // 2, D), lambda i: (i // (HB // BLK),
                                                  i % (HB // BLK), 0)),
        pl.BlockSpec((BLK, D), lambda i: (i, 0)),
        pl.BlockSpec((NC, BLK), lambda i: (0, i)),
        pl.BlockSpec((D,), lambda i: (0,)),
    ],
    out_specs=pl.BlockSpec((BLK, D), lambda i: (i, 0)),
    out_shape=jax.ShapeDtypeStruct((NP, D), jnp.float32),
)

_PI = np.zeros((D,), dtype=np.int32)
for _g in range(4):
    for _i in range(16):
        _PI[32 * _g + _i] = 16 * _g + _i
        _PI[32 * _g + 16 + _i] = 64 + 16 * _g + _i


def kernel(x, index, W, b):
    idx32 = index.astype(jnp.int32)
    src = idx32[0]
    dst = idx32[1]

    dst0 = jnp.concatenate(
        [dst, jnp.full((EP0 - E,), TRASH0, jnp.int32)]).reshape(NW, NCH0, CH)

    src1 = jnp.concatenate(
        [src, jnp.zeros((EP1 - E,), jnp.int32)]).reshape(NS, NCH1, CH)
    dstp = jnp.concatenate([dst, jnp.full((EP1 - E,), -1, jnp.int32)])
    trash = HB + (jnp.arange(EP1, dtype=jnp.int32) % 128)
    los = jnp.arange(NC, dtype=jnp.int32) * HB
    local = dstp[None, :] - los[:, None]
    dstl = jnp.where((local >= 0) & (local < HB), local,
                     trash[None, :]).reshape(NC, NS, NCH1, CH)

    xp = jnp.pad(x, ((0, NP - N), (0, 0)))
    Wp = W[:, _PI]

    degp = _deg_kernel(dst0)
    hn, hnp = _dense(xp, W, Wp, degp)
    agg = _agg_kernel(src1, dstl, hnp)
    out = _final(agg, hn, degp, b)
    return (out[:N], index)
